# Initial kernel scaffold; baseline (speedup 1.0000x reference)
import functools

import jax
import jax.numpy as jnp
from jax import lax
from jax.experimental import pallas as pl
from jax.experimental.pallas import tpu as pltpu
from jax.experimental.pallas import tpu_sc as plsc

N = 10000
E = 320000
D_IN = 128
D_HID = 128
D_OUT = 64

NC = 2
NS = 16
NW = NC * NS
CH = 128
NCH = 80
E_PAD = NW * NCH * CH
N_ACC = N + 16
ROWS_PER_TILE_ACC = N_ACC // NS
ROWS_PER_TILE_OUT = N // NS

_mesh = plsc.VectorSubcoreMesh(core_axis_name="c", subcore_axis_name="s")


def _fill_rows(buf, value, nrows, ncols):
    vec = jnp.full((16,), value, jnp.float32)

    def body(r, _):
        for c0 in range(ncols // 16):
            buf[r, pl.ds(c0 * 16, 16)] = vec
        return 0

    lax.fori_loop(0, nrows, body, 0)


@functools.partial(
    pl.kernel,
    out_type=jax.ShapeDtypeStruct((NC, N, 16), jnp.float32),
    mesh=_mesh,
    scratch_types=[
        pltpu.VMEM((NCH, CH), jnp.int32),
        pltpu.VMEM((CH, 16), jnp.float32),
        pltpu.VMEM_SHARED((N_ACC, 16), jnp.float32),
    ],
)
def _deg_kernel(dst_hbm, out_hbm, idx_v, buf, acc):
    c = lax.axis_index("c")
    s = lax.axis_index("s")
    w = c * NS + s
    pltpu.sync_copy(dst_hbm.at[pl.ds(w * NCH, NCH)], idx_v)
    _fill_rows(buf, 0.0, CH, 16)
    base = s * ROWS_PER_TILE_ACC
    for k in range(4):
        pltpu.sync_copy(buf, acc.at[pl.ds(base + k * CH, CH)])
    pltpu.sync_copy(buf.at[pl.ds(0, ROWS_PER_TILE_ACC - 4 * CH)],
                    acc.at[pl.ds(base + 4 * CH, ROWS_PER_TILE_ACC - 4 * CH)])
    _fill_rows(buf, 1.0, CH, 16)
    plsc.subcore_barrier()

    def body(j, _):
        pltpu.sync_copy(buf, acc.at[idx_v.at[j]], add=True)
        return 0

    lax.fori_loop(0, NCH, body, 0)
    plsc.subcore_barrier()
    pltpu.sync_copy(acc.at[pl.ds(s * ROWS_PER_TILE_OUT, ROWS_PER_TILE_OUT)],
                    out_hbm.at[c, pl.ds(s * ROWS_PER_TILE_OUT, ROWS_PER_TILE_OUT)])


def _make_edge_kernel(d):
    @functools.partial(
        pl.kernel,
        out_type=jax.ShapeDtypeStruct((NC, N, d), jnp.float32),
        mesh=_mesh,
        scratch_types=[
            pltpu.VMEM((NCH, CH), jnp.int32),
            pltpu.VMEM((NCH, CH), jnp.int32),
            pltpu.VMEM((CH, d), jnp.float32),
            pltpu.VMEM_SHARED((N_ACC, d), jnp.float32),
            pltpu.SemaphoreType.DMA,
        ],
    )
    def edge_kernel(tab_hbm, src_hbm, dst_hbm, out_hbm, src_v, dst_v, buf, acc, sem):
        c = lax.axis_index("c")
        s = lax.axis_index("s")
        w = c * NS + s
        pltpu.sync_copy(src_hbm.at[pl.ds(w * NCH, NCH)], src_v)
        pltpu.sync_copy(dst_hbm.at[pl.ds(w * NCH, NCH)], dst_v)
        _fill_rows(buf, 0.0, CH, d)
        base = s * ROWS_PER_TILE_ACC
        for k in range(4):
            pltpu.sync_copy(buf, acc.at[pl.ds(base + k * CH, CH)])
        pltpu.sync_copy(buf.at[pl.ds(0, ROWS_PER_TILE_ACC - 4 * CH)],
                        acc.at[pl.ds(base + 4 * CH, ROWS_PER_TILE_ACC - 4 * CH)])
        plsc.subcore_barrier()

        def body(j, _):
            pltpu.async_copy(tab_hbm.at[src_v.at[j]], buf, sem).wait()
            pltpu.sync_copy(buf, acc.at[dst_v.at[j]], add=True)
            return 0

        lax.fori_loop(0, NCH, body, 0)
        plsc.subcore_barrier()
        pltpu.sync_copy(acc.at[pl.ds(s * ROWS_PER_TILE_OUT, ROWS_PER_TILE_OUT)],
                        out_hbm.at[c, pl.ds(s * ROWS_PER_TILE_OUT, ROWS_PER_TILE_OUT)])

    return edge_kernel


_edge_kernel_128 = _make_edge_kernel(D_HID)
_edge_kernel_64 = _make_edge_kernel(D_OUT)


_BLK = 400
_GRID = N // _BLK


def _tc1_body(deg_ref, x_ref, w_ref, dis_ref, hp_ref):
    deg = deg_ref[0] + deg_ref[1] + 1.0
    dis = lax.rsqrt(deg)
    dis_ref[...] = dis
    h = jnp.dot(x_ref[...], w_ref[...], preferred_element_type=jnp.float32)
    hp_ref[...] = h * dis[:, 0:1]


def _tc_mid_body(acc_ref, hp_ref, dis_ref, b_ref, w_ref, out_ref):
    dis = dis_ref[...][:, 0:1]
    t = (acc_ref[0] + acc_ref[1] + hp_ref[...]) * dis + b_ref[...]
    t = jnp.maximum(t, 0.0)
    out_ref[...] = jnp.dot(t, w_ref[...], preferred_element_type=jnp.float32) * dis


def _tc_final_body(acc_ref, hp_ref, dis_ref, b_ref, out_ref):
    dis = dis_ref[...][:, 0:1]
    out_ref[...] = (acc_ref[0] + acc_ref[1] + hp_ref[...]) * dis + b_ref[...]


def _tc1(deg, x, w):
    return pl.pallas_call(
        _tc1_body,
        grid=(_GRID,),
        in_specs=[
            pl.BlockSpec((NC, _BLK, 16), lambda i: (0, i, 0)),
            pl.BlockSpec((_BLK, D_IN), lambda i: (i, 0)),
            pl.BlockSpec((D_IN, D_HID), lambda i: (0, 0)),
        ],
        out_specs=[
            pl.BlockSpec((_BLK, 16), lambda i: (i, 0)),
            pl.BlockSpec((_BLK, D_HID), lambda i: (i, 0)),
        ],
        out_shape=[
            jax.ShapeDtypeStruct((N, 16), jnp.float32),
            jax.ShapeDtypeStruct((N, D_HID), jnp.float32),
        ],
    )(deg, x, w)


def _tc_mid(acc, hp, dis, b, w):
    d_in, d_out = w.shape
    return pl.pallas_call(
        _tc_mid_body,
        grid=(_GRID,),
        in_specs=[
            pl.BlockSpec((NC, _BLK, d_in), lambda i: (0, i, 0)),
            pl.BlockSpec((_BLK, d_in), lambda i: (i, 0)),
            pl.BlockSpec((_BLK, 16), lambda i: (i, 0)),
            pl.BlockSpec((1, d_in), lambda i: (0, 0)),
            pl.BlockSpec((d_in, d_out), lambda i: (0, 0)),
        ],
        out_specs=pl.BlockSpec((_BLK, d_out), lambda i: (i, 0)),
        out_shape=jax.ShapeDtypeStruct((N, d_out), jnp.float32),
    )(acc, hp, dis, b, w)


def _tc_final(acc, hp, dis, b):
    d = hp.shape[1]
    return pl.pallas_call(
        _tc_final_body,
        grid=(_GRID,),
        in_specs=[
            pl.BlockSpec((NC, _BLK, d), lambda i: (0, i, 0)),
            pl.BlockSpec((_BLK, d), lambda i: (i, 0)),
            pl.BlockSpec((_BLK, 16), lambda i: (i, 0)),
            pl.BlockSpec((1, d), lambda i: (0, 0)),
        ],
        out_specs=pl.BlockSpec((_BLK, d), lambda i: (i, 0)),
        out_shape=jax.ShapeDtypeStruct((N, d), jnp.float32),
    )(acc, hp, dis, b)


def kernel(x, edge_index, W1, b1, W2, b2, W_out, b_out):
    pad = E_PAD - E
    src = jnp.concatenate([edge_index[0], jnp.zeros((pad,), jnp.int32)])
    dst = jnp.concatenate([edge_index[1], jnp.full((pad,), N, jnp.int32)])
    src = src.reshape(NW * NCH, CH)
    dst = dst.reshape(NW * NCH, CH)

    deg = _deg_kernel(dst)
    dis, h1p = _tc1(deg, x, W1)
    acc1 = _edge_kernel_128(h1p, src, dst)
    h2p = _tc_mid(acc1, h1p, dis, b1.reshape(1, D_HID), W2)
    acc2 = _edge_kernel_128(h2p, src, dst)
    h3p = _tc_mid(acc2, h2p, dis, b2.reshape(1, D_HID), W_out)
    acc3 = _edge_kernel_64(h3p, src, dst)
    out = _tc_final(acc3, h3p, dis, b_out.reshape(1, D_OUT))
    return out


# speedup vs baseline: 6.8653x; 6.8653x over previous
import functools

import jax
import jax.numpy as jnp
from jax import lax
from jax.experimental import pallas as pl
from jax.experimental.pallas import tpu as pltpu
from jax.experimental.pallas import tpu_sc as plsc

N = 10000
E = 320000
D_IN = 128
D_HID = 128
D_OUT = 64

NC = 2
NS = 16
NW = NC * NS
CH = 128
NCH = 80
E_PAD = NW * NCH * CH
N_ACC = 10112
ROWS_PER_TILE_ACC = N_ACC // NS

_mesh = plsc.VectorSubcoreMesh(core_axis_name="c", subcore_axis_name="s")


def _fill_rows(buf, value, nrows, ncols):
    vec = jnp.full((16,), value, jnp.float32)

    def body(r, _):
        for c0 in range(ncols // 16):
            buf[r, pl.ds(c0 * 16, 16)] = vec
        return 0

    lax.fori_loop(0, nrows, body, 0)


@functools.partial(
    pl.kernel,
    out_type=jax.ShapeDtypeStruct((NC, N_ACC, 16), jnp.float32),
    mesh=_mesh,
    scratch_types=[
        pltpu.VMEM((NCH, CH), jnp.int32),
        pltpu.VMEM((CH, 16), jnp.float32),
        pltpu.VMEM_SHARED((N_ACC, 16), jnp.float32),
    ],
)
def _deg_kernel(dst_hbm, out_hbm, idx_v, buf, acc):
    c = lax.axis_index("c")
    s = lax.axis_index("s")
    w = c * NS + s
    pltpu.sync_copy(dst_hbm.at[pl.ds(w * NCH, NCH)], idx_v)
    _fill_rows(buf, 0.0, CH, 16)
    base = s * ROWS_PER_TILE_ACC
    for k in range(4):
        pltpu.sync_copy(buf, acc.at[pl.ds(base + k * CH, CH)])
    pltpu.sync_copy(buf.at[pl.ds(0, ROWS_PER_TILE_ACC - 4 * CH)],
                    acc.at[pl.ds(base + 4 * CH, ROWS_PER_TILE_ACC - 4 * CH)])
    _fill_rows(buf, 1.0, CH, 16)
    plsc.subcore_barrier()

    def body(j, _):
        pltpu.sync_copy(buf, acc.at[idx_v.at[j]], add=True)
        return 0

    lax.fori_loop(0, NCH, body, 0)
    plsc.subcore_barrier()
    pltpu.sync_copy(acc.at[pl.ds(base, ROWS_PER_TILE_ACC)],
                    out_hbm.at[c, pl.ds(base, ROWS_PER_TILE_ACC)])


def _make_edge_kernel(d):
    @functools.partial(
        pl.kernel,
        out_type=jax.ShapeDtypeStruct((NC, N_ACC, d), jnp.float32),
        mesh=_mesh,
        scratch_types=[
            pltpu.VMEM((NCH, CH), jnp.int32),
            pltpu.VMEM((NCH, CH), jnp.int32),
            pltpu.VMEM((CH, d), jnp.float32),
            pltpu.VMEM_SHARED((N_ACC, d), jnp.float32),
            pltpu.SemaphoreType.DMA,
        ],
    )
    def edge_kernel(tab_hbm, src_hbm, dst_hbm, out_hbm, src_v, dst_v, buf, acc, sem):
        c = lax.axis_index("c")
        s = lax.axis_index("s")
        w = c * NS + s
        pltpu.sync_copy(src_hbm.at[pl.ds(w * NCH, NCH)], src_v)
        pltpu.sync_copy(dst_hbm.at[pl.ds(w * NCH, NCH)], dst_v)
        _fill_rows(buf, 0.0, CH, d)
        base = s * ROWS_PER_TILE_ACC
        for k in range(4):
            pltpu.sync_copy(buf, acc.at[pl.ds(base + k * CH, CH)])
        pltpu.sync_copy(buf.at[pl.ds(0, ROWS_PER_TILE_ACC - 4 * CH)],
                        acc.at[pl.ds(base + 4 * CH, ROWS_PER_TILE_ACC - 4 * CH)])
        plsc.subcore_barrier()

        def body(j, _):
            pltpu.async_copy(tab_hbm.at[src_v.at[j]], buf, sem).wait()
            pltpu.sync_copy(buf, acc.at[dst_v.at[j]], add=True)
            return 0

        lax.fori_loop(0, NCH, body, 0)
        plsc.subcore_barrier()
        pltpu.sync_copy(acc.at[pl.ds(base, ROWS_PER_TILE_ACC)],
                        out_hbm.at[c, pl.ds(base, ROWS_PER_TILE_ACC)])

    return edge_kernel


_edge_kernel_128 = _make_edge_kernel(D_HID)


_BLK = 400
_GRID = N // _BLK


def _tc1_body(deg_ref, x_ref, w_ref, dis_ref, hp_ref):
    deg = deg_ref[0] + deg_ref[1] + 1.0
    dis = lax.rsqrt(deg)
    dis_ref[...] = dis
    h = jnp.dot(x_ref[...], w_ref[...], preferred_element_type=jnp.float32)
    hp_ref[...] = h * dis[:, 0:1]


def _tc_mid_body(acc_ref, hp_ref, dis_ref, b_ref, w_ref, out_ref):
    dis = dis_ref[...][:, 0:1]
    t = (acc_ref[0] + acc_ref[1] + hp_ref[...]) * dis + b_ref[...]
    t = jnp.maximum(t, 0.0)
    out_ref[...] = jnp.dot(t, w_ref[...], preferred_element_type=jnp.float32) * dis


def _tc_final_body(acc_ref, hp_ref, dis_ref, b_ref, out_ref):
    dis = dis_ref[...][:, 0:1]
    out_ref[...] = (acc_ref[0] + acc_ref[1] + hp_ref[...]) * dis + b_ref[...]


def _tc1(deg, x, w):
    return pl.pallas_call(
        _tc1_body,
        grid=(_GRID,),
        in_specs=[
            pl.BlockSpec((NC, _BLK, 16), lambda i: (0, i, 0)),
            pl.BlockSpec((_BLK, D_IN), lambda i: (i, 0)),
            pl.BlockSpec((D_IN, D_HID), lambda i: (0, 0)),
        ],
        out_specs=[
            pl.BlockSpec((_BLK, 16), lambda i: (i, 0)),
            pl.BlockSpec((_BLK, D_HID), lambda i: (i, 0)),
        ],
        out_shape=[
            jax.ShapeDtypeStruct((N, 16), jnp.float32),
            jax.ShapeDtypeStruct((N, D_HID), jnp.float32),
        ],
    )(deg, x, w)


def _tc_mid(acc, hp, dis, b, w):
    d_in, d_out = w.shape
    return pl.pallas_call(
        _tc_mid_body,
        grid=(_GRID,),
        in_specs=[
            pl.BlockSpec((NC, _BLK, d_in), lambda i: (0, i, 0)),
            pl.BlockSpec((_BLK, d_in), lambda i: (i, 0)),
            pl.BlockSpec((_BLK, 16), lambda i: (i, 0)),
            pl.BlockSpec((1, d_in), lambda i: (0, 0)),
            pl.BlockSpec((d_in, d_out), lambda i: (0, 0)),
        ],
        out_specs=pl.BlockSpec((_BLK, d_out), lambda i: (i, 0)),
        out_shape=jax.ShapeDtypeStruct((N, d_out), jnp.float32),
    )(acc, hp, dis, b, w)


def _tc_final(acc, hp, dis, b):
    d = hp.shape[1]
    return pl.pallas_call(
        _tc_final_body,
        grid=(_GRID,),
        in_specs=[
            pl.BlockSpec((NC, _BLK, d), lambda i: (0, i, 0)),
            pl.BlockSpec((_BLK, d), lambda i: (i, 0)),
            pl.BlockSpec((_BLK, 16), lambda i: (i, 0)),
            pl.BlockSpec((1, d), lambda i: (0, 0)),
        ],
        out_specs=pl.BlockSpec((_BLK, d), lambda i: (i, 0)),
        out_shape=jax.ShapeDtypeStruct((N, d), jnp.float32),
    )(acc, hp, dis, b)


def kernel(x, edge_index, W1, b1, W2, b2, W_out, b_out):
    pad = E_PAD - E
    src = jnp.concatenate([edge_index[0], jnp.zeros((pad,), jnp.int32)])
    dst = jnp.concatenate([edge_index[1], jnp.full((pad,), N, jnp.int32)])
    src = src.reshape(NW * NCH, CH)
    dst = dst.reshape(NW * NCH, CH)

    deg = _deg_kernel(dst)[:, :N]
    dis, h1p = _tc1(deg, x, W1)
    acc1 = _edge_kernel_128(h1p, src, dst)[:, :N]
    h2p = _tc_mid(acc1, h1p, dis, b1.reshape(1, D_HID), W2)
    acc2 = _edge_kernel_128(h2p, src, dst)[:, :N]
    h3p = _tc_mid(acc2, h2p, dis, b2.reshape(1, D_HID), W_out)
    h3p_pad = jnp.pad(h3p, ((0, 0), (0, D_HID - D_OUT)))
    acc3 = _edge_kernel_128(h3p_pad, src, dst)[:, :N, :D_OUT]
    out = _tc_final(acc3, h3p, dis, b_out.reshape(1, D_OUT))
    return out


# speedup vs baseline: 17.3702x; 2.5302x over previous
import functools

import jax
import jax.numpy as jnp
from jax import lax
from jax.experimental import pallas as pl
from jax.experimental.pallas import tpu as pltpu
from jax.experimental.pallas import tpu_sc as plsc

N = 10000
E = 320000
D_IN = 128
D_HID = 128
D_OUT = 64

NC = 2
NS = 16
NW = NC * NS
CH = 128
NCH = 80
E_PAD = NW * NCH * CH
N_ACC = 10112
ROWS_PER_TILE_ACC = N_ACC // NS

_mesh = plsc.VectorSubcoreMesh(core_axis_name="c", subcore_axis_name="s")


def _fill_rows(buf, value, nrows, ncols):
    vec = jnp.full((16,), value, jnp.float32)

    def body(r, _):
        for c0 in range(ncols // 16):
            buf[r, pl.ds(c0 * 16, 16)] = vec
        return 0

    lax.fori_loop(0, nrows, body, 0)


@functools.partial(
    pl.kernel,
    out_type=jax.ShapeDtypeStruct((NC, N_ACC, 16), jnp.float32),
    mesh=_mesh,
    scratch_types=[
        pltpu.VMEM((NCH, CH), jnp.int32),
        pltpu.VMEM((CH, 16), jnp.float32),
        pltpu.VMEM_SHARED((N_ACC, 16), jnp.float32),
    ],
)
def _deg_kernel(dst_hbm, out_hbm, idx_v, buf, acc):
    c = lax.axis_index("c")
    s = lax.axis_index("s")
    w = c * NS + s
    pltpu.sync_copy(dst_hbm.at[pl.ds(w * NCH, NCH)], idx_v)
    _fill_rows(buf, 0.0, CH, 16)
    base = s * ROWS_PER_TILE_ACC
    for k in range(4):
        pltpu.sync_copy(buf, acc.at[pl.ds(base + k * CH, CH)])
    pltpu.sync_copy(buf.at[pl.ds(0, ROWS_PER_TILE_ACC - 4 * CH)],
                    acc.at[pl.ds(base + 4 * CH, ROWS_PER_TILE_ACC - 4 * CH)])
    _fill_rows(buf, 1.0, CH, 16)
    plsc.subcore_barrier()

    def body(j, _):
        pltpu.sync_copy(buf, acc.at[idx_v.at[j]], add=True)
        return 0

    lax.fori_loop(0, NCH, body, 0)
    plsc.subcore_barrier()
    pltpu.sync_copy(acc.at[pl.ds(base, ROWS_PER_TILE_ACC)],
                    out_hbm.at[c, pl.ds(base, ROWS_PER_TILE_ACC)])


def _make_edge_kernel(d):
    @functools.partial(
        pl.kernel,
        out_type=jax.ShapeDtypeStruct((NC, N_ACC, d), jnp.float32),
        mesh=_mesh,
        scratch_types=[
            pltpu.VMEM((NCH, CH), jnp.int32),
            pltpu.VMEM((NCH, CH), jnp.int32),
            pltpu.VMEM((CH, d), jnp.float32),
            pltpu.VMEM_SHARED((N_ACC, d), jnp.float32),
            pltpu.SemaphoreType.DMA,
        ],
    )
    def edge_kernel(tab_hbm, src_hbm, dst_hbm, out_hbm, src_v, dst_v, buf, acc, sem):
        c = lax.axis_index("c")
        s = lax.axis_index("s")
        w = c * NS + s
        pltpu.sync_copy(src_hbm.at[pl.ds(w * NCH, NCH)], src_v)
        pltpu.sync_copy(dst_hbm.at[pl.ds(w * NCH, NCH)], dst_v)
        _fill_rows(buf, 0.0, CH, d)
        base = s * ROWS_PER_TILE_ACC
        for k in range(4):
            pltpu.sync_copy(buf, acc.at[pl.ds(base + k * CH, CH)])
        pltpu.sync_copy(buf.at[pl.ds(0, ROWS_PER_TILE_ACC - 4 * CH)],
                        acc.at[pl.ds(base + 4 * CH, ROWS_PER_TILE_ACC - 4 * CH)])
        plsc.subcore_barrier()

        def body(j, _):
            pltpu.async_copy(tab_hbm.at[src_v.at[j]], buf, sem).wait()
            pltpu.sync_copy(buf, acc.at[dst_v.at[j]], add=True)
            return 0

        lax.fori_loop(0, NCH, body, 0)
        plsc.subcore_barrier()
        pltpu.sync_copy(acc.at[pl.ds(base, ROWS_PER_TILE_ACC)],
                        out_hbm.at[c, pl.ds(base, ROWS_PER_TILE_ACC)])

    return edge_kernel


_edge_kernel_128 = _make_edge_kernel(D_HID)


_BLK = 400
_GRID = N // _BLK


def _tc1_body(deg_ref, x_ref, w_ref, dis_ref, hp_ref):
    deg = deg_ref[0] + deg_ref[1] + 1.0
    dis = lax.rsqrt(deg)
    dis_ref[...] = dis
    h = jnp.dot(x_ref[...], w_ref[...], preferred_element_type=jnp.float32)
    hp_ref[...] = h * dis[:, 0:1]


def _tc_mid_body(acc_ref, hp_ref, dis_ref, b_ref, w_ref, out_ref):
    dis = dis_ref[...][:, 0:1]
    t = (acc_ref[0] + acc_ref[1] + hp_ref[...]) * dis + b_ref[...]
    t = jnp.maximum(t, 0.0)
    out_ref[...] = jnp.dot(t, w_ref[...], preferred_element_type=jnp.float32) * dis


def _tc_final_body(acc_ref, hp_ref, dis_ref, b_ref, out_ref):
    dis = dis_ref[...][:, 0:1]
    out_ref[...] = (acc_ref[0] + acc_ref[1] + hp_ref[...]) * dis + b_ref[...]


def _tc1(deg, x, w):
    return pl.pallas_call(
        _tc1_body,
        grid=(_GRID,),
        in_specs=[
            pl.BlockSpec((NC, _BLK, 16), lambda i: (0, i, 0)),
            pl.BlockSpec((_BLK, D_IN), lambda i: (i, 0)),
            pl.BlockSpec((D_IN, D_HID), lambda i: (0, 0)),
        ],
        out_specs=[
            pl.BlockSpec((_BLK, 16), lambda i: (i, 0)),
            pl.BlockSpec((_BLK, D_HID), lambda i: (i, 0)),
        ],
        out_shape=[
            jax.ShapeDtypeStruct((N, 16), jnp.float32),
            jax.ShapeDtypeStruct((N, D_HID), jnp.float32),
        ],
    )(deg, x, w)


def _tc_mid(acc, hp, dis, b, w):
    d_in, d_out = w.shape
    return pl.pallas_call(
        _tc_mid_body,
        grid=(_GRID,),
        in_specs=[
            pl.BlockSpec((NC, _BLK, d_in), lambda i: (0, i, 0)),
            pl.BlockSpec((_BLK, d_in), lambda i: (i, 0)),
            pl.BlockSpec((_BLK, 16), lambda i: (i, 0)),
            pl.BlockSpec((1, d_in), lambda i: (0, 0)),
            pl.BlockSpec((d_in, d_out), lambda i: (0, 0)),
        ],
        out_specs=pl.BlockSpec((_BLK, d_out), lambda i: (i, 0)),
        out_shape=jax.ShapeDtypeStruct((N, d_out), jnp.float32),
    )(acc, hp, dis, b, w)


def _tc_final(acc, hp, dis, b):
    d = hp.shape[1]
    return pl.pallas_call(
        _tc_final_body,
        grid=(_GRID,),
        in_specs=[
            pl.BlockSpec((NC, _BLK, d), lambda i: (0, i, 0)),
            pl.BlockSpec((_BLK, d), lambda i: (i, 0)),
            pl.BlockSpec((_BLK, 16), lambda i: (i, 0)),
            pl.BlockSpec((1, d), lambda i: (0, 0)),
        ],
        out_specs=pl.BlockSpec((_BLK, d), lambda i: (i, 0)),
        out_shape=jax.ShapeDtypeStruct((N, d), jnp.float32),
    )(acc, hp, dis, b)


def kernel(x, edge_index, W1, b1, W2, b2, W_out, b_out):
    pad = E_PAD - E
    pad_i = jnp.arange(pad, dtype=jnp.int32)
    src = jnp.concatenate([edge_index[0], pad_i % N])
    dst = jnp.concatenate([edge_index[1], N + pad_i % (N_ACC - N)])
    src = src.reshape(NW * NCH, CH)
    dst = dst.reshape(NW * NCH, CH)

    deg = _deg_kernel(dst)[:, :N]
    dis, h1p = _tc1(deg, x, W1)
    acc1 = _edge_kernel_128(h1p, src, dst)[:, :N]
    h2p = _tc_mid(acc1, h1p, dis, b1.reshape(1, D_HID), W2)
    acc2 = _edge_kernel_128(h2p, src, dst)[:, :N]
    h3p = _tc_mid(acc2, h2p, dis, b2.reshape(1, D_HID), W_out)
    h3p_pad = jnp.pad(h3p, ((0, 0), (0, D_HID - D_OUT)))
    acc3 = _edge_kernel_128(h3p_pad, src, dst)[:, :N, :D_OUT]
    out = _tc_final(acc3, h3p, dis, b_out.reshape(1, D_OUT))
    return out
